# baseline (speedup 1.0000x reference)
import jax
import jax.numpy as jnp
from jax import lax
from jax.experimental import pallas as pl
from jax.experimental.pallas import tpu as pltpu
from jax.experimental.pallas import tpu_sc as plsc

VOCAB = 100000
EMBED = 64
BATCH = 4096
SEQLEN = 200
LN_EPS = 1e-5

NC, NS = 2, 16
NW = NC * NS
SEQ_W = BATCH // NW
CSEQ = 4
CROWS = CSEQ * SEQLEN
NCHUNK = SEQ_W // CSEQ
NVEC = EMBED // 16
GSPLIT = ((0, 128), (128, SEQLEN - 128))


def _body(x_hbm, tok_hbm, pos_hbm, g_hbm, b_hbm, out_hbm,
          idx_a, idx_b, rows_a, rows_b, pos_v, g_v, b_v,
          gsem_a, gsem_b, wsem_a, wsem_b):
    cid = lax.axis_index("c")
    sid = lax.axis_index("s")
    wid = sid * NC + cid
    seq0 = wid * SEQ_W

    pltpu.sync_copy(pos_hbm, pos_v)
    pltpu.sync_copy(g_hbm, g_v)
    pltpu.sync_copy(b_hbm, b_v)
    gamma = [g_v[pl.ds(16 * j, 16)] for j in range(NVEC)]
    beta = [b_v[pl.ds(16 * j, 16)] for j in range(NVEC)]

    def gather_descs(idx_x, rows_x, gsem_x):
        return [pltpu.make_async_copy(
                    tok_hbm.at[idx_x.at[pl.ds(s * SEQLEN + off, n)]],
                    rows_x.at[s, pl.ds(off, n)],
                    gsem_x)
                for s in range(CSEQ) for off, n in GSPLIT]

    def start_gather(k, idx_x, rows_x, gsem_x):
        pltpu.sync_copy(
            x_hbm.at[pl.ds((seq0 + k * CSEQ) * SEQLEN, CROWS)], idx_x)
        for d in gather_descs(idx_x, rows_x, gsem_x):
            d.start()

    def write_desc(k, rows_x, wsem_x):
        return pltpu.make_async_copy(
            rows_x, out_hbm.at[pl.ds(seq0 + k * CSEQ, CSEQ)], wsem_x)

    inv_d = 1.0 / EMBED

    def compute(rows_x):
        @plsc.parallel_loop(0, SEQLEN, unroll=4)
        def _row(p):
            pv = [pos_v[pl.ds(p * EMBED + 16 * j, 16)] for j in range(NVEC)]
            for s in range(CSEQ):
                h = [rows_x[s, p, pl.ds(16 * j, 16)] + pv[j]
                     for j in range(NVEC)]
                s1 = jnp.sum((h[0] + h[1]) + (h[2] + h[3]))
                s2 = jnp.sum((h[0] * h[0] + h[1] * h[1])
                             + (h[2] * h[2] + h[3] * h[3]))
                mean = s1 * inv_d
                var = s2 * inv_d - mean * mean
                xv = var + LN_EPS
                i = lax.bitcast_convert_type(xv, jnp.int32)
                i = 0x5F3759DF - lax.shift_right_logical(i, 1)
                y = lax.bitcast_convert_type(i, jnp.float32)
                hx = 0.5 * xv
                y = y * (1.5 - hx * y * y)
                y = y * (1.5 - hx * y * y)
                y = y * (1.5 - hx * y * y)
                for j in range(NVEC):
                    rows_x[s, p, pl.ds(16 * j, 16)] = (
                        ((h[j] - mean) * y) * gamma[j] + beta[j])

    bufs = ((idx_a, rows_a, gsem_a, wsem_a),
            (idx_b, rows_b, gsem_b, wsem_b))

    start_gather(0, idx_a, rows_a, gsem_a)

    @pl.loop(0, NCHUNK, step=2)
    def _chunks(c):
        for b in range(2):
            idx_x, rows_x, gsem_x, wsem_x = bufs[b]
            idx_y, rows_y, gsem_y, wsem_y = bufs[1 - b]
            k = c + b

            @pl.when(k + 1 < NCHUNK)
            def _():
                @pl.when(k >= 1)
                def _():
                    write_desc(0, rows_y, wsem_y).wait()
                start_gather(k + 1, idx_y, rows_y, gsem_y)

            for d in gather_descs(idx_x, rows_x, gsem_x):
                d.wait()
            compute(rows_x)
            write_desc(k, rows_x, wsem_x).start()

    write_desc(0, rows_a, wsem_a).wait()
    write_desc(0, rows_b, wsem_b).wait()


_sc_call = pl.kernel(
    _body,
    out_type=jax.ShapeDtypeStruct((BATCH, SEQLEN, EMBED), jnp.float32),
    mesh=plsc.VectorSubcoreMesh(core_axis_name="c", subcore_axis_name="s"),
    scratch_types=[
        pltpu.VMEM((CROWS,), jnp.int32),
        pltpu.VMEM((CROWS,), jnp.int32),
        pltpu.VMEM((CSEQ, SEQLEN, EMBED), jnp.float32),
        pltpu.VMEM((CSEQ, SEQLEN, EMBED), jnp.float32),
        pltpu.VMEM((SEQLEN * EMBED,), jnp.float32),
        pltpu.VMEM((EMBED,), jnp.float32),
        pltpu.VMEM((EMBED,), jnp.float32),
        pltpu.SemaphoreType.DMA,
        pltpu.SemaphoreType.DMA,
        pltpu.SemaphoreType.DMA,
        pltpu.SemaphoreType.DMA,
    ],
    compiler_params=pltpu.CompilerParams(needs_layout_passes=False,
                                         use_tc_tiling_on_sc=False),
)


def kernel(x, token_table, pos_table, ln_gamma, ln_beta):
    x_flat = x.reshape(BATCH * SEQLEN)
    pos_flat = pos_table[:SEQLEN].reshape(SEQLEN * EMBED)
    out = _sc_call(x_flat, token_table, pos_flat, ln_gamma, ln_beta)
    return out, x > 0


# speedup vs baseline: 1.0291x; 1.0291x over previous
import jax
import jax.numpy as jnp
from jax import lax
from jax.experimental import pallas as pl
from jax.experimental.pallas import tpu as pltpu
from jax.experimental.pallas import tpu_sc as plsc

VOCAB = 100000
EMBED = 64
BATCH = 4096
SEQLEN = 200
LN_EPS = 1e-5

NC, NS = 2, 16
NW = NC * NS
SEQ_W = BATCH // NW
CSEQ = 2
CROWS = CSEQ * SEQLEN
NCHUNK = SEQ_W // CSEQ
NBUF = 4
NVEC = EMBED // 16
GSPLIT = ((0, 128), (128, SEQLEN - 128))


def _body(x_hbm, tok_hbm, pos_hbm, g_hbm, b_hbm, out_hbm, *refs):
    idx_v = refs[0:NBUF]
    rows_v = refs[NBUF:2 * NBUF]
    pos_v, g_v, b_v = refs[2 * NBUF:2 * NBUF + 3]
    gsem = refs[2 * NBUF + 3:2 * NBUF + 3 + NBUF]
    wsem = refs[2 * NBUF + 3 + NBUF:]

    cid = lax.axis_index("c")
    sid = lax.axis_index("s")
    wid = sid * NC + cid
    seq0 = wid * SEQ_W

    pltpu.sync_copy(pos_hbm, pos_v)
    pltpu.sync_copy(g_hbm, g_v)
    pltpu.sync_copy(b_hbm, b_v)
    gamma = [g_v[pl.ds(16 * j, 16)] for j in range(NVEC)]
    beta = [b_v[pl.ds(16 * j, 16)] for j in range(NVEC)]

    def gather_descs(m):
        return [pltpu.make_async_copy(
                    tok_hbm.at[idx_v[m].at[pl.ds(s * SEQLEN + off, n)]],
                    rows_v[m].at[s, pl.ds(off, n)],
                    gsem[m])
                for s in range(CSEQ) for off, n in GSPLIT]

    def start_gather(k, m):
        pltpu.sync_copy(
            x_hbm.at[pl.ds((seq0 + k * CSEQ) * SEQLEN, CROWS)], idx_v[m])
        for d in gather_descs(m):
            d.start()

    def write_desc(k, m):
        return pltpu.make_async_copy(
            rows_v[m], out_hbm.at[pl.ds(seq0 + k * CSEQ, CSEQ)], wsem[m])

    inv_d = 1.0 / EMBED

    def compute(m):
        rows_x = rows_v[m]

        @plsc.parallel_loop(0, SEQLEN, unroll=2)
        def _row(p):
            pv = [pos_v[pl.ds(p * EMBED + 16 * j, 16)] for j in range(NVEC)]
            for s in range(CSEQ):
                h = [rows_x[s, p, pl.ds(16 * j, 16)] + pv[j]
                     for j in range(NVEC)]
                s1 = jnp.sum((h[0] + h[1]) + (h[2] + h[3]))
                s2 = jnp.sum((h[0] * h[0] + h[1] * h[1])
                             + (h[2] * h[2] + h[3] * h[3]))
                mean = s1 * inv_d
                var = s2 * inv_d - mean * mean
                xv = var + LN_EPS
                i = lax.bitcast_convert_type(xv, jnp.int32)
                i = 0x5F3759DF - lax.shift_right_logical(i, 1)
                y = lax.bitcast_convert_type(i, jnp.float32)
                hx = 0.5 * xv
                y = y * (1.5 - hx * y * y)
                y = y * (1.5 - hx * y * y)
                y = y * (1.5 - hx * y * y)
                for j in range(NVEC):
                    rows_x[s, p, pl.ds(16 * j, 16)] = (
                        ((h[j] - mean) * y) * gamma[j] + beta[j])

    for m in range(NBUF - 1):
        start_gather(m, m)

    @pl.loop(0, NCHUNK, step=NBUF)
    def _chunks(c):
        for b in range(NBUF):
            k = c + b
            kpre = k + NBUF - 1
            mpre = (b + NBUF - 1) % NBUF

            @pl.when(kpre < NCHUNK)
            def _():
                @pl.when(kpre >= NBUF)
                def _():
                    write_desc(0, mpre).wait()
                start_gather(kpre, mpre)

            for d in gather_descs(b):
                d.wait()
            compute(b)
            write_desc(k, b).start()

    for m in range(NBUF):
        write_desc(0, m).wait()


_sc_call = pl.kernel(
    _body,
    out_type=jax.ShapeDtypeStruct((BATCH, SEQLEN, EMBED), jnp.float32),
    mesh=plsc.VectorSubcoreMesh(core_axis_name="c", subcore_axis_name="s"),
    scratch_types=(
        [pltpu.VMEM((CROWS,), jnp.int32) for _ in range(NBUF)]
        + [pltpu.VMEM((CSEQ, SEQLEN, EMBED), jnp.float32)
           for _ in range(NBUF)]
        + [pltpu.VMEM((SEQLEN * EMBED,), jnp.float32),
           pltpu.VMEM((EMBED,), jnp.float32),
           pltpu.VMEM((EMBED,), jnp.float32)]
        + [pltpu.SemaphoreType.DMA for _ in range(2 * NBUF)]
    ),
    compiler_params=pltpu.CompilerParams(needs_layout_passes=False,
                                         use_tc_tiling_on_sc=False),
)


def kernel(x, token_table, pos_table, ln_gamma, ln_beta):
    x_flat = x.reshape(BATCH * SEQLEN)
    pos_flat = pos_table[:SEQLEN].reshape(SEQLEN * EMBED)
    out = _sc_call(x_flat, token_table, pos_flat, ln_gamma, ln_beta)
    return out, x > 0


# speedup vs baseline: 1.0861x; 1.0553x over previous
import jax
import jax.numpy as jnp
from jax import lax
from jax.experimental import pallas as pl
from jax.experimental.pallas import tpu as pltpu
from jax.experimental.pallas import tpu_sc as plsc

VOCAB = 100000
EMBED = 64
BATCH = 4096
SEQLEN = 200
LN_EPS = 1e-5

NC, NS = 2, 16
NW = NC * NS
SEQ_W = BATCH // NW
CSEQ = 2
CROWS = CSEQ * SEQLEN
CELEMS = CROWS * EMBED
NCHUNK = SEQ_W // CSEQ
NVEC = EMBED // 16
GSPLIT = ((0, 128), (128, SEQLEN - 128))


def _body(x_hbm, tok_hbm, pos_hbm, g_hbm, b_hbm, out_hbm,
          idx_a, idx_b, rows_a, rows_b, res_a, res_b, pos_v, g_v, b_v,
          gsem_a, gsem_b, wsem_a, wsem_b):
    idx_v = (idx_a, idx_b)
    rows_v = (rows_a, rows_b)
    res_v = (res_a, res_b)
    gsem = (gsem_a, gsem_b)
    wsem = (wsem_a, wsem_b)

    cid = lax.axis_index("c")
    sid = lax.axis_index("s")
    wid = sid * NC + cid
    seq0 = wid * SEQ_W

    pltpu.sync_copy(pos_hbm, pos_v)
    pltpu.sync_copy(g_hbm, g_v)
    pltpu.sync_copy(b_hbm, b_v)
    gamma = [g_v[pl.ds(16 * j, 16)] for j in range(NVEC)]
    beta = [b_v[pl.ds(16 * j, 16)] for j in range(NVEC)]

    def gather_descs(m):
        return [pltpu.make_async_copy(
                    tok_hbm.at[idx_v[m].at[pl.ds(s * SEQLEN + off, n)]],
                    rows_v[m].at[pl.ds(s * SEQLEN + off, n)],
                    gsem[m])
                for s in range(CSEQ) for off, n in GSPLIT]

    def start_gather(k, m):
        pltpu.sync_copy(
            x_hbm.at[pl.ds((seq0 + k * CSEQ) * SEQLEN, CROWS)], idx_v[m])
        for d in gather_descs(m):
            d.start()

    def write_desc(k, m):
        return pltpu.make_async_copy(
            res_v[m],
            out_hbm.at[pl.ds((seq0 + k * CSEQ) * SEQLEN * EMBED, CELEMS)],
            wsem[m])

    inv_d = 1.0 / EMBED

    def compute(m):
        rows_x = rows_v[m]
        res_x = res_v[m]

        @plsc.parallel_loop(0, SEQLEN, unroll=2)
        def _row(p):
            pv = [pos_v[pl.ds(p * EMBED + 16 * j, 16)] for j in range(NVEC)]
            for s in range(CSEQ):
                r = s * SEQLEN + p
                h = [rows_x[r, pl.ds(16 * j, 16)] + pv[j]
                     for j in range(NVEC)]
                s1 = jnp.sum((h[0] + h[1]) + (h[2] + h[3]))
                s2 = jnp.sum((h[0] * h[0] + h[1] * h[1])
                             + (h[2] * h[2] + h[3] * h[3]))
                mean = s1 * inv_d
                var = s2 * inv_d - mean * mean
                xv = var + LN_EPS
                i = lax.bitcast_convert_type(xv, jnp.int32)
                i = 0x5F3759DF - lax.shift_right_logical(i, 1)
                y = lax.bitcast_convert_type(i, jnp.float32)
                hx = 0.5 * xv
                y = y * (1.5 - hx * y * y)
                y = y * (1.5 - hx * y * y)
                y = y * (1.5 - hx * y * y)
                for j in range(NVEC):
                    res_x[pl.ds(r * EMBED + 16 * j, 16)] = (
                        ((h[j] - mean) * y) * gamma[j] + beta[j])

    start_gather(0, 0)

    @pl.loop(0, NCHUNK, step=2)
    def _chunks(c):
        for b in range(2):
            m = b
            k = c + b

            @pl.when(k + 1 < NCHUNK)
            def _():
                start_gather(k + 1, 1 - m)

            for d in gather_descs(m):
                d.wait()

            @pl.when(k >= 2)
            def _():
                write_desc(0, m).wait()
            compute(m)
            write_desc(k, m).start()

    write_desc(0, 0).wait()
    write_desc(0, 1).wait()


_sc_call = pl.kernel(
    _body,
    out_type=jax.ShapeDtypeStruct((BATCH * SEQLEN * EMBED,), jnp.float32),
    mesh=plsc.VectorSubcoreMesh(core_axis_name="c", subcore_axis_name="s"),
    scratch_types=[
        pltpu.VMEM((CROWS,), jnp.int32),
        pltpu.VMEM((CROWS,), jnp.int32),
        pltpu.VMEM((CROWS, EMBED), jnp.float32),
        pltpu.VMEM((CROWS, EMBED), jnp.float32),
        pltpu.VMEM((CELEMS,), jnp.float32),
        pltpu.VMEM((CELEMS,), jnp.float32),
        pltpu.VMEM((SEQLEN * EMBED,), jnp.float32),
        pltpu.VMEM((EMBED,), jnp.float32),
        pltpu.VMEM((EMBED,), jnp.float32),
        pltpu.SemaphoreType.DMA,
        pltpu.SemaphoreType.DMA,
        pltpu.SemaphoreType.DMA,
        pltpu.SemaphoreType.DMA,
    ],
    compiler_params=pltpu.CompilerParams(needs_layout_passes=False,
                                         use_tc_tiling_on_sc=False),
)


def kernel(x, token_table, pos_table, ln_gamma, ln_beta):
    x_flat = x.reshape(BATCH * SEQLEN)
    pos_flat = pos_table[:SEQLEN].reshape(SEQLEN * EMBED)
    out = _sc_call(x_flat, token_table, pos_flat, ln_gamma, ln_beta)
    return out.reshape(BATCH, SEQLEN, EMBED), x > 0
